# baseline (speedup 1.0000x reference)
import functools

import jax
import jax.numpy as jnp
from jax import lax
from jax.experimental import pallas as pl
from jax.experimental.pallas import tpu as pltpu
from jax.experimental.pallas import tpu_sc as plsc

B = 16384
D = 64
W = 128
NC = 2
NS = 16
NW = NC * NS
CHUNK = B // NW
NIDX = 128
NJ = CHUNK // NIDX
LANES = 16
WAVE = 256
NWAVE = CHUNK // WAVE
JPW = WAVE // NIDX


def _mf_body(uid_hbm, iid_hbm, p_hbm, q_hbm, out_hbm,
             g_u, g_i, h_u, h_i, rows_u, rows_i, m_v, out_v, sem):
    c = lax.axis_index("c")
    s = lax.axis_index("s")
    wid = s * NC + c
    base = wid * NJ

    pltpu.sync_copy(uid_hbm.at[pl.ds(base, NJ)], g_u)
    pltpu.sync_copy(iid_hbm.at[pl.ds(base, NJ)], g_i)

    n_u = 2 * p_hbm.shape[0]
    n_i = 2 * q_hbm.shape[0]

    for a in range(NJ):
        for k in range(NIDX // LANES):
            sl = pl.ds(k * LANES, LANES)
            fl = pl.ds(a * NIDX + k * LANES, LANES)
            u = g_u[a, sl]
            ru = jnp.where(u == 0, n_u - 1, u - 1)
            g_u[a, sl] = jnp.right_shift(ru, 1)
            h_u[fl] = jnp.bitwise_and(ru, 1)
            v = g_i[a, sl]
            ri = jnp.where(v == 0, n_i - 1, v - 1)
            g_i[a, sl] = jnp.right_shift(ri, 1)
            h_i[fl] = jnp.bitwise_and(ri, 1)

    zero16 = jnp.zeros((LANES,), jnp.int32)
    col0 = lax.iota(jnp.int32, LANES) * LANES

    for w in range(NWAVE):
        cps = []
        for j in range(JPW):
            cj = w * JPW + j
            cps.append(pltpu.async_copy(
                p_hbm.at[g_u.at[cj]], rows_u.at[pl.ds(j * NIDX, NIDX)], sem))
            cps.append(pltpu.async_copy(
                q_hbm.at[g_i.at[cj]], rows_i.at[pl.ds(j * NIDX, NIDX)], sem))
        for cp in cps:
            cp.wait()

        def group(g16, carry):
            for l in range(LANES):
                r = g16 * LANES + l
                gidx = w * WAVE + r + zero16
                hu = plsc.load_gather(h_u, [gidx])
                hi = plsc.load_gather(h_i, [gidx])
                acc = None
                for jj in range(D // LANES):
                    lo = pl.ds(jj * LANES, LANES)
                    hi_sl = pl.ds(D + jj * LANES, LANES)
                    us = jnp.where(hu == 0, rows_u[r, lo], rows_u[r, hi_sl])
                    vs = jnp.where(hi == 0, rows_i[r, lo], rows_i[r, hi_sl])
                    acc = us * vs if acc is None else acc + us * vs
                m_v[pl.ds(l * LANES, LANES)] = acc
            res = plsc.load_gather(m_v, [col0])
            for l in range(1, LANES):
                res = res + plsc.load_gather(m_v, [col0 + l])
            out_v[pl.ds(w * WAVE + g16 * LANES, LANES)] = res
            return carry

        lax.fori_loop(0, WAVE // LANES, group, 0)

    pltpu.sync_copy(out_v, out_hbm.at[pl.ds(wid * CHUNK, CHUNK)])


@functools.partial(
    pl.kernel,
    mesh=plsc.VectorSubcoreMesh(core_axis_name="c", subcore_axis_name="s"),
    out_type=jax.ShapeDtypeStruct((B,), jnp.float32),
    compiler_params=pltpu.CompilerParams(
        needs_layout_passes=False, use_tc_tiling_on_sc=True),
    scratch_types=[
        pltpu.VMEM((NJ, NIDX), jnp.int32),
        pltpu.VMEM((NJ, NIDX), jnp.int32),
        pltpu.VMEM((CHUNK,), jnp.int32),
        pltpu.VMEM((CHUNK,), jnp.int32),
        pltpu.VMEM((WAVE, W), jnp.float32),
        pltpu.VMEM((WAVE, W), jnp.float32),
        pltpu.VMEM((LANES * LANES,), jnp.float32),
        pltpu.VMEM((CHUNK,), jnp.float32),
        pltpu.SemaphoreType.DMA,
    ],
)
def _mf_sc(uid_hbm, iid_hbm, p_hbm, q_hbm, out_hbm,
           g_u, g_i, h_u, h_i, rows_u, rows_i, m_v, out_v, sem):
    _mf_body(uid_hbm, iid_hbm, p_hbm, q_hbm, out_hbm,
             g_u, g_i, h_u, h_i, rows_u, rows_i, m_v, out_v, sem)


def kernel(avg_score, user_id, item_id, p, q):
    del avg_score
    uid2 = user_id.reshape(B // NIDX, NIDX)
    iid2 = item_id.reshape(B // NIDX, NIDX)
    pr = p.reshape(p.shape[0] // 2, W)
    qr = q.reshape(q.shape[0] // 2, W)
    out = _mf_sc(uid2, iid2, pr, qr)
    return out.reshape(B, 1, 1)


# speedup vs baseline: 1.0697x; 1.0697x over previous
import functools

import jax
import jax.numpy as jnp
from jax import lax
from jax.experimental import pallas as pl
from jax.experimental.pallas import tpu as pltpu
from jax.experimental.pallas import tpu_sc as plsc

B = 16384
D = 64
W = 128
NC = 2
NS = 16
NW = NC * NS
CHUNK = B // NW
NIDX = 128
NJ = CHUNK // NIDX
LANES = 16
WAVE = 256
NWAVE = CHUNK // WAVE
JPW = WAVE // NIDX


def _mf_body(uid_hbm, iid_hbm, p_hbm, q_hbm, out_hbm,
             idx_u, idx_i, rows_u, rows_i, m_v, out_v, sem):
    c = lax.axis_index("c")
    s = lax.axis_index("s")
    wid = s * NC + c
    base = wid * NJ

    pltpu.sync_copy(uid_hbm.at[pl.ds(base, NJ)], idx_u)
    pltpu.sync_copy(iid_hbm.at[pl.ds(base, NJ)], idx_i)

    last_u = p_hbm.shape[0] - 1
    last_i = q_hbm.shape[0] - 1

    for a in range(NJ):
        for k in range(NIDX // LANES):
            sl = pl.ds(k * LANES, LANES)
            u = idx_u[a, sl]
            idx_u[a, sl] = jnp.where(u == 0, last_u, u - 1)
            v = idx_i[a, sl]
            idx_i[a, sl] = jnp.where(v == 0, last_i, v - 1)

    col0 = lax.iota(jnp.int32, LANES) * LANES

    for w in range(NWAVE):
        cps = []
        for j in range(JPW):
            cj = w * JPW + j
            cps.append(pltpu.async_copy(
                p_hbm.at[idx_u.at[cj]], rows_u.at[pl.ds(j * NIDX, NIDX)], sem))
            cps.append(pltpu.async_copy(
                q_hbm.at[idx_i.at[cj]], rows_i.at[pl.ds(j * NIDX, NIDX)], sem))
        for cp in cps:
            cp.wait()

        def group(g, carry):
            for l in range(LANES):
                r = g * LANES + l
                acc = rows_u[r, pl.ds(0, LANES)] * rows_i[r, pl.ds(0, LANES)]
                for jj in range(1, D // LANES):
                    sl = pl.ds(jj * LANES, LANES)
                    acc = acc + rows_u[r, sl] * rows_i[r, sl]
                m_v[pl.ds(l * LANES, LANES)] = acc
            res = plsc.load_gather(m_v, [col0])
            for l in range(1, LANES):
                res = res + plsc.load_gather(m_v, [col0 + l])
            out_v[pl.ds(w * WAVE + g * LANES, LANES)] = res
            return carry

        lax.fori_loop(0, WAVE // LANES, group, 0)

    pltpu.sync_copy(out_v, out_hbm.at[pl.ds(wid * CHUNK, CHUNK)])


@functools.partial(
    pl.kernel,
    mesh=plsc.VectorSubcoreMesh(core_axis_name="c", subcore_axis_name="s"),
    out_type=jax.ShapeDtypeStruct((B,), jnp.float32),
    compiler_params=pltpu.CompilerParams(
        needs_layout_passes=False, use_tc_tiling_on_sc=True),
    scratch_types=[
        pltpu.VMEM((NJ, NIDX), jnp.int32),
        pltpu.VMEM((NJ, NIDX), jnp.int32),
        pltpu.VMEM((WAVE, W), jnp.float32),
        pltpu.VMEM((WAVE, W), jnp.float32),
        pltpu.VMEM((LANES * LANES,), jnp.float32),
        pltpu.VMEM((CHUNK,), jnp.float32),
        pltpu.SemaphoreType.DMA,
    ],
)
def _mf_sc(uid_hbm, iid_hbm, p_hbm, q_hbm, out_hbm,
           idx_u, idx_i, rows_u, rows_i, m_v, out_v, sem):
    _mf_body(uid_hbm, iid_hbm, p_hbm, q_hbm, out_hbm,
             idx_u, idx_i, rows_u, rows_i, m_v, out_v, sem)


def kernel(avg_score, user_id, item_id, p, q):
    del avg_score
    uid2 = user_id.reshape(B // NIDX, NIDX)
    iid2 = item_id.reshape(B // NIDX, NIDX)
    pp = jnp.pad(p, ((0, 0), (0, W - D)))
    qp = jnp.pad(q, ((0, 0), (0, W - D)))
    out = _mf_sc(uid2, iid2, pp, qp)
    return out.reshape(B, 1, 1)


# speedup vs baseline: 1.2147x; 1.1356x over previous
import functools

import jax
import jax.numpy as jnp
from jax import lax
from jax.experimental import pallas as pl
from jax.experimental.pallas import tpu as pltpu
from jax.experimental.pallas import tpu_sc as plsc

B = 16384
D = 64
W = 128
NC = 2
NS = 16
NW = NC * NS
CHUNK = B // NW
NIDX = 128
NJ = CHUNK // NIDX
LANES = 16
WAVE = 256
NWAVE = CHUNK // WAVE
JPW = WAVE // NIDX


def _mf_body(uid_hbm, iid_hbm, p_hbm, out_hbm,
             idx_u, idx_i, rows_u, rows_i, m_v, out_v, sem):
    c = lax.axis_index("c")
    s = lax.axis_index("s")
    wid = s * NC + c
    base = wid * NJ

    pltpu.sync_copy(uid_hbm.at[pl.ds(base, NJ)], idx_u)
    pltpu.sync_copy(iid_hbm.at[pl.ds(base, NJ)], idx_i)

    last_u = p_hbm.shape[0] - 1
    last_i = p_hbm.shape[0] - 1

    for a in range(NJ):
        for k in range(NIDX // LANES):
            sl = pl.ds(k * LANES, LANES)
            u = idx_u[a, sl]
            idx_u[a, sl] = jnp.where(u == 0, last_u, u - 1)
            v = idx_i[a, sl]
            idx_i[a, sl] = jnp.where(v == 0, last_i, v - 1)

    col0 = lax.iota(jnp.int32, LANES) * LANES

    for w in range(NWAVE):
        cps = []
        for j in range(JPW):
            cj = w * JPW + j
            cps.append(pltpu.async_copy(
                p_hbm.at[idx_u.at[cj]], rows_u.at[pl.ds(j * NIDX, NIDX)], sem))
            cps.append(pltpu.async_copy(
                p_hbm.at[idx_i.at[cj]], rows_i.at[pl.ds(j * NIDX, NIDX)], sem))
        for cp in cps:
            cp.wait()

        def group(g, carry):
            for l in range(LANES):
                r = g * LANES + l
                acc = (rows_u[r, pl.ds(0, LANES)]
                       * rows_i[r, pl.ds(D, LANES)])
                for jj in range(1, D // LANES):
                    acc = acc + (rows_u[r, pl.ds(jj * LANES, LANES)]
                                 * rows_i[r, pl.ds(D + jj * LANES, LANES)])
                m_v[pl.ds(l * LANES, LANES)] = acc
            res = plsc.load_gather(m_v, [col0])
            for l in range(1, LANES):
                res = res + plsc.load_gather(m_v, [col0 + l])
            out_v[pl.ds(w * WAVE + g * LANES, LANES)] = res
            return carry

        lax.fori_loop(0, WAVE // LANES, group, 0)

    pltpu.sync_copy(out_v, out_hbm.at[pl.ds(wid * CHUNK, CHUNK)])


@functools.partial(
    pl.kernel,
    mesh=plsc.VectorSubcoreMesh(core_axis_name="c", subcore_axis_name="s"),
    out_type=jax.ShapeDtypeStruct((B,), jnp.float32),
    compiler_params=pltpu.CompilerParams(
        needs_layout_passes=False, use_tc_tiling_on_sc=True),
    scratch_types=[
        pltpu.VMEM((NJ, NIDX), jnp.int32),
        pltpu.VMEM((NJ, NIDX), jnp.int32),
        pltpu.VMEM((WAVE, W), jnp.float32),
        pltpu.VMEM((WAVE, W), jnp.float32),
        pltpu.VMEM((LANES * LANES,), jnp.float32),
        pltpu.VMEM((CHUNK,), jnp.float32),
        pltpu.SemaphoreType.DMA,
    ],
)
def _mf_sc(uid_hbm, iid_hbm, p_hbm, out_hbm,
           idx_u, idx_i, rows_u, rows_i, m_v, out_v, sem):
    _mf_body(uid_hbm, iid_hbm, p_hbm, out_hbm,
             idx_u, idx_i, rows_u, rows_i, m_v, out_v, sem)


def kernel(avg_score, user_id, item_id, p, q):
    del avg_score
    uid2 = user_id.reshape(B // NIDX, NIDX)
    iid2 = item_id.reshape(B // NIDX, NIDX)
    t = jnp.concatenate([p, q], axis=1)
    out = _mf_sc(uid2, iid2, t)
    return out.reshape(B, 1, 1)
